# baseline (device time: 26901 ns/iter reference)
import jax
import jax.numpy as jnp
from jax import lax
from jax.experimental import pallas as pl
from jax.experimental.pallas import tpu as pltpu

N_DEV = 4
B = 2
SQ = 128
H_LOC = 4
DH = 64
CHUNK = H_LOC * DH


def kernel(x, Wq, K_ext, V_ext, Wo):
    d_model = x.shape[-1]

    def body(x_ref, wq_ref, k_ref, v_ref, wo_ref, out_ref,
             comm_ref, send_sems, recv_sems):
        my_pos = lax.axis_index("i")
        right = lax.rem(my_pos + 1, N_DEV)
        left = lax.rem(my_pos + N_DEV - 1, N_DEV)

        for b in range(B):
            q = jnp.dot(
                x_ref[b],
                wq_ref[:, pl.ds(my_pos * CHUNK, CHUNK)],
                preferred_element_type=jnp.float32,
            )
            for h in range(H_LOC):
                qh = q[:, h * DH:(h + 1) * DH]
                kh = k_ref[b, :, h, :]
                s = lax.dot_general(
                    qh, kh, (((1,), (1,)), ((), ())),
                    preferred_element_type=jnp.float32,
                ) * 0.125
                s = s - jnp.max(s, axis=-1, keepdims=True)
                w = jnp.exp(s)
                w = w / jnp.sum(w, axis=-1, keepdims=True)
                ctx = jnp.dot(
                    w, v_ref[b, :, h, :],
                    preferred_element_type=jnp.float32,
                )
                comm_ref[0, b, :, h * DH:(h + 1) * DH] = ctx

        barrier_sem = pltpu.get_barrier_semaphore()
        for nbr in (left, right):
            pl.semaphore_signal(
                barrier_sem, inc=1,
                device_id=(nbr,), device_id_type=pl.DeviceIdType.MESH,
            )
        pl.semaphore_wait(barrier_sem, 2)

        def hop(h):
            return pltpu.make_async_remote_copy(
                src_ref=comm_ref.at[h],
                dst_ref=comm_ref.at[h + 1],
                send_sem=send_sems.at[h],
                recv_sem=recv_sems.at[h],
                device_id=(right,),
                device_id_type=pl.DeviceIdType.MESH,
            )

        rdmas = [hop(0)]
        rdmas[0].start()

        for b in range(B):
            out_ref[b] = jnp.dot(
                comm_ref[0, b],
                wo_ref[pl.ds(my_pos * CHUNK, CHUNK), :],
                preferred_element_type=jnp.float32,
            )

        for h in range(N_DEV - 1):
            rdmas[h].wait_recv()
            if h < N_DEV - 2:
                nxt = hop(h + 1)
                nxt.start()
                rdmas.append(nxt)
            origin = lax.rem(my_pos + N_DEV - (h + 1), N_DEV)
            for b in range(B):
                out_ref[b] += jnp.dot(
                    comm_ref[h + 1, b],
                    wo_ref[pl.ds(origin * CHUNK, CHUNK), :],
                    preferred_element_type=jnp.float32,
                )

        for r in rdmas:
            r.wait_send()

    return pl.pallas_call(
        body,
        out_shape=jax.ShapeDtypeStruct((B, SQ, d_model), jnp.float32),
        in_specs=[pl.BlockSpec(memory_space=pltpu.VMEM)] * 5,
        out_specs=pl.BlockSpec(memory_space=pltpu.VMEM),
        scratch_shapes=[
            pltpu.VMEM((N_DEV, B, SQ, CHUNK), jnp.float32),
            pltpu.SemaphoreType.DMA((N_DEV - 1,)),
            pltpu.SemaphoreType.DMA((N_DEV - 1,)),
        ],
        compiler_params=pltpu.CompilerParams(collective_id=0),
    )(x, Wq, K_ext, V_ext, Wo)


# device time: 21096 ns/iter; 1.2752x vs baseline; 1.2752x over previous
import jax
import jax.numpy as jnp
from jax import lax
from jax.experimental import pallas as pl
from jax.experimental.pallas import tpu as pltpu

N_DEV = 4
B = 2
SQ = 128
H_LOC = 4
DH = 64
CHUNK = H_LOC * DH


def kernel(x, Wq, K_ext, V_ext, Wo):
    d_model = x.shape[-1]

    def body(x_ref, wq_ref, k_ref, v_ref, wo_ref, out_ref,
             ctx_ref, comm_ref, send_sems, recv_sems):
        my_pos = lax.axis_index("i")

        xf = jnp.reshape(x_ref[:, :, :], (B * SQ, d_model))
        q = jnp.dot(
            xf, wq_ref[:, pl.ds(my_pos * CHUNK, CHUNK)],
            preferred_element_type=jnp.float32,
        )
        for b in range(B):
            qb = q[b * SQ:(b + 1) * SQ, :]
            for h in range(H_LOC):
                qh = qb[:, h * DH:(h + 1) * DH]
                kh = k_ref[b, :, h, :]
                s = lax.dot_general(
                    qh, kh, (((1,), (1,)), ((), ())),
                    preferred_element_type=jnp.float32,
                ) * 0.125
                s = s - jnp.max(s, axis=-1, keepdims=True)
                w = jnp.exp(s)
                w = w / jnp.sum(w, axis=-1, keepdims=True)
                ctx = jnp.dot(
                    w, v_ref[b, :, h, :],
                    preferred_element_type=jnp.float32,
                )
                ctx_ref[b, :, h * DH:(h + 1) * DH] = ctx

        barrier_sem = pltpu.get_barrier_semaphore()
        for j in range(1, N_DEV):
            pl.semaphore_signal(
                barrier_sem, inc=1,
                device_id=(lax.rem(my_pos + j, N_DEV),),
                device_id_type=pl.DeviceIdType.MESH,
            )
        pl.semaphore_wait(barrier_sem, N_DEV - 1)

        sends = []
        for j in range(1, N_DEV):
            r = pltpu.make_async_remote_copy(
                src_ref=ctx_ref,
                dst_ref=comm_ref.at[N_DEV - 1 - j],
                send_sem=send_sems.at[j - 1],
                recv_sem=recv_sems.at[N_DEV - 1 - j],
                device_id=(lax.rem(my_pos + j, N_DEV),),
                device_id_type=pl.DeviceIdType.MESH,
            )
            r.start()
            sends.append(r)

        cf = jnp.reshape(ctx_ref[:, :], (B * SQ, CHUNK))
        acc = jnp.dot(
            cf, wo_ref[pl.ds(my_pos * CHUNK, CHUNK), :],
            preferred_element_type=jnp.float32,
        )

        for s in (0, 2, 1):
            recv = pltpu.make_async_remote_copy(
                src_ref=ctx_ref,
                dst_ref=comm_ref.at[s],
                send_sem=send_sems.at[0],
                recv_sem=recv_sems.at[s],
                device_id=(my_pos,),
                device_id_type=pl.DeviceIdType.MESH,
            )
            recv.wait_recv()
            origin = lax.rem(my_pos + s + 1, N_DEV)
            chunk = jnp.reshape(comm_ref[s, :, :, :], (B * SQ, CHUNK))
            acc = acc + jnp.dot(
                chunk, wo_ref[pl.ds(origin * CHUNK, CHUNK), :],
                preferred_element_type=jnp.float32,
            )

        out_ref[:, :, :] = jnp.reshape(acc, (B, SQ, d_model))

        for r in sends:
            r.wait_send()

    return pl.pallas_call(
        body,
        out_shape=jax.ShapeDtypeStruct((B, SQ, d_model), jnp.float32),
        in_specs=[pl.BlockSpec(memory_space=pltpu.VMEM)] * 5,
        out_specs=pl.BlockSpec(memory_space=pltpu.VMEM),
        scratch_shapes=[
            pltpu.VMEM((B, SQ, CHUNK), jnp.float32),
            pltpu.VMEM((N_DEV - 1, B, SQ, CHUNK), jnp.float32),
            pltpu.SemaphoreType.DMA((N_DEV - 1,)),
            pltpu.SemaphoreType.DMA((N_DEV - 1,)),
        ],
        compiler_params=pltpu.CompilerParams(collective_id=0),
    )(x, Wq, K_ext, V_ext, Wo)


# device time: 18507 ns/iter; 1.4536x vs baseline; 1.1399x over previous
import jax
import jax.numpy as jnp
from jax import lax
from jax.experimental import pallas as pl
from jax.experimental.pallas import tpu as pltpu

N_DEV = 4
B = 2
SQ = 128
H_LOC = 4
DH = 64
CHUNK = H_LOC * DH


def kernel(x, Wq, K_ext, V_ext, Wo):
    d_model = x.shape[-1]

    def body(x_ref, wq_ref, k_ref, v_ref, wo_ref, out_ref,
             ctx_ref, comm_ref, send_sems, recv_sems):
        my_pos = lax.axis_index("i")

        xf = jnp.reshape(x_ref[:, :, :], (B * SQ, d_model)).astype(jnp.bfloat16)
        wq = wq_ref[:, pl.ds(my_pos * CHUNK, CHUNK)].astype(jnp.bfloat16)
        q = jnp.dot(xf, wq, preferred_element_type=jnp.float32)
        for b in range(B):
            qb = q[b * SQ:(b + 1) * SQ, :].astype(jnp.bfloat16)
            for h in range(H_LOC):
                qh = qb[:, h * DH:(h + 1) * DH]
                kh = k_ref[b, :, h, :].astype(jnp.bfloat16)
                s = lax.dot_general(
                    qh, kh, (((1,), (1,)), ((), ())),
                    preferred_element_type=jnp.float32,
                ) * 0.125
                s = s - jnp.max(s, axis=-1, keepdims=True)
                w = jnp.exp(s)
                w = (w / jnp.sum(w, axis=-1, keepdims=True)).astype(jnp.bfloat16)
                ctx = jnp.dot(
                    w, v_ref[b, :, h, :].astype(jnp.bfloat16),
                    preferred_element_type=jnp.float32,
                )
                ctx_ref[b, :, h * DH:(h + 1) * DH] = ctx.astype(jnp.bfloat16)

        barrier_sem = pltpu.get_barrier_semaphore()
        for j in range(1, N_DEV):
            pl.semaphore_signal(
                barrier_sem, inc=1,
                device_id=(lax.rem(my_pos + j, N_DEV),),
                device_id_type=pl.DeviceIdType.MESH,
            )
        pl.semaphore_wait(barrier_sem, N_DEV - 1)

        sends = []
        for j in range(1, N_DEV):
            r = pltpu.make_async_remote_copy(
                src_ref=ctx_ref,
                dst_ref=comm_ref.at[N_DEV - 1 - j],
                send_sem=send_sems.at[j - 1],
                recv_sem=recv_sems.at[N_DEV - 1 - j],
                device_id=(lax.rem(my_pos + j, N_DEV),),
                device_id_type=pl.DeviceIdType.MESH,
            )
            r.start()
            sends.append(r)

        cf = jnp.reshape(ctx_ref[:, :, :], (B * SQ, CHUNK))
        acc = jnp.dot(
            cf,
            wo_ref[pl.ds(my_pos * CHUNK, CHUNK), :].astype(jnp.bfloat16),
            preferred_element_type=jnp.float32,
        )

        for s in (0, 2, 1):
            recv = pltpu.make_async_remote_copy(
                src_ref=ctx_ref,
                dst_ref=comm_ref.at[s],
                send_sem=send_sems.at[0],
                recv_sem=recv_sems.at[s],
                device_id=(my_pos,),
                device_id_type=pl.DeviceIdType.MESH,
            )
            recv.wait_recv()
            origin = lax.rem(my_pos + s + 1, N_DEV)
            chunk = jnp.reshape(comm_ref[s, :, :, :], (B * SQ, CHUNK))
            acc = acc + jnp.dot(
                chunk,
                wo_ref[pl.ds(origin * CHUNK, CHUNK), :].astype(jnp.bfloat16),
                preferred_element_type=jnp.float32,
            )

        out_ref[:, :, :] = jnp.reshape(acc, (B, SQ, d_model))

        for r in sends:
            r.wait_send()

    return pl.pallas_call(
        body,
        out_shape=jax.ShapeDtypeStruct((B, SQ, d_model), jnp.float32),
        in_specs=[pl.BlockSpec(memory_space=pltpu.VMEM)] * 5,
        out_specs=pl.BlockSpec(memory_space=pltpu.VMEM),
        scratch_shapes=[
            pltpu.VMEM((B, SQ, CHUNK), jnp.bfloat16),
            pltpu.VMEM((N_DEV - 1, B, SQ, CHUNK), jnp.bfloat16),
            pltpu.SemaphoreType.DMA((N_DEV - 1,)),
            pltpu.SemaphoreType.DMA((N_DEV - 1,)),
        ],
        compiler_params=pltpu.CompilerParams(collective_id=0),
    )(x, Wq, K_ext, V_ext, Wo)


# device time: 16692 ns/iter; 1.6116x vs baseline; 1.1087x over previous
import jax
import jax.numpy as jnp
from jax import lax
from jax.experimental import pallas as pl
from jax.experimental.pallas import tpu as pltpu

N_DEV = 4
B = 2
SQ = 128
H_LOC = 4
DH = 64
CHUNK = H_LOC * DH


def kernel(x, Wq, K_ext, V_ext, Wo):
    d_model = x.shape[-1]
    K_ext = K_ext.reshape(B, SQ, CHUNK)
    V_ext = V_ext.reshape(B, SQ, CHUNK)

    def body(x_ref, wq_ref, k_ref, v_ref, wo_ref, out_ref,
             ctx_ref, comm_ref, send_sems, recv_sems):
        my_pos = lax.axis_index("i")

        xf = jnp.reshape(x_ref[:, :, :], (B * SQ, d_model)).astype(jnp.bfloat16)
        wq = wq_ref[:, pl.ds(my_pos * CHUNK, CHUNK)].astype(jnp.bfloat16)
        q = jnp.dot(xf, wq, preferred_element_type=jnp.float32)
        q = q.astype(jnp.bfloat16)

        blocks = []
        for b in range(B):
            kb = k_ref[b, :, :].astype(jnp.bfloat16)
            for h in range(H_LOC):
                qh = q[b * SQ:(b + 1) * SQ, h * DH:(h + 1) * DH]
                kh = kb[:, h * DH:(h + 1) * DH]
                blocks.append(lax.dot_general(
                    qh, kh, (((1,), (1,)), ((), ())),
                    preferred_element_type=jnp.float32,
                ))
        s = jnp.concatenate(blocks, axis=0) * 0.125
        s = s - jnp.max(s, axis=-1, keepdims=True)
        w = jnp.exp(s)
        w = (w / jnp.sum(w, axis=-1, keepdims=True)).astype(jnp.bfloat16)

        for b in range(B):
            vb = v_ref[b, :, :].astype(jnp.bfloat16)
            for h in range(H_LOC):
                i = b * H_LOC + h
                ctx = jnp.dot(
                    w[i * SQ:(i + 1) * SQ, :],
                    vb[:, h * DH:(h + 1) * DH],
                    preferred_element_type=jnp.float32,
                )
                ctx_ref[b, :, h * DH:(h + 1) * DH] = ctx.astype(jnp.bfloat16)

        barrier_sem = pltpu.get_barrier_semaphore()
        for j in range(1, N_DEV):
            pl.semaphore_signal(
                barrier_sem, inc=1,
                device_id=(lax.rem(my_pos + j, N_DEV),),
                device_id_type=pl.DeviceIdType.MESH,
            )
        pl.semaphore_wait(barrier_sem, N_DEV - 1)

        sends = []
        for j in range(1, N_DEV):
            r = pltpu.make_async_remote_copy(
                src_ref=ctx_ref,
                dst_ref=comm_ref.at[N_DEV - 1 - j],
                send_sem=send_sems.at[j - 1],
                recv_sem=recv_sems.at[N_DEV - 1 - j],
                device_id=(lax.rem(my_pos + j, N_DEV),),
                device_id_type=pl.DeviceIdType.MESH,
            )
            r.start()
            sends.append(r)

        cf = jnp.reshape(ctx_ref[:, :, :], (B * SQ, CHUNK))
        acc = jnp.dot(
            cf,
            wo_ref[pl.ds(my_pos * CHUNK, CHUNK), :].astype(jnp.bfloat16),
            preferred_element_type=jnp.float32,
        )

        for s in (0, 2, 1):
            recv = pltpu.make_async_remote_copy(
                src_ref=ctx_ref,
                dst_ref=comm_ref.at[s],
                send_sem=send_sems.at[0],
                recv_sem=recv_sems.at[s],
                device_id=(my_pos,),
                device_id_type=pl.DeviceIdType.MESH,
            )
            recv.wait_recv()
            origin = lax.rem(my_pos + s + 1, N_DEV)
            chunk = jnp.reshape(comm_ref[s, :, :, :], (B * SQ, CHUNK))
            acc = acc + jnp.dot(
                chunk,
                wo_ref[pl.ds(origin * CHUNK, CHUNK), :].astype(jnp.bfloat16),
                preferred_element_type=jnp.float32,
            )

        out_ref[:, :, :] = jnp.reshape(acc, (B, SQ, d_model))

        for r in sends:
            r.wait_send()

    return pl.pallas_call(
        body,
        out_shape=jax.ShapeDtypeStruct((B, SQ, d_model), jnp.float32),
        in_specs=[pl.BlockSpec(memory_space=pltpu.VMEM)] * 5,
        out_specs=pl.BlockSpec(memory_space=pltpu.VMEM),
        scratch_shapes=[
            pltpu.VMEM((B, SQ, CHUNK), jnp.bfloat16),
            pltpu.VMEM((N_DEV - 1, B, SQ, CHUNK), jnp.bfloat16),
            pltpu.SemaphoreType.DMA((N_DEV - 1,)),
            pltpu.SemaphoreType.DMA((N_DEV - 1,)),
        ],
        compiler_params=pltpu.CompilerParams(collective_id=0),
    )(x, Wq, K_ext, V_ext, Wo)


# device time: 15640 ns/iter; 1.7200x vs baseline; 1.0673x over previous
import os

import jax
import jax.numpy as jnp
from jax import lax
from jax.experimental import pallas as pl
from jax.experimental.pallas import tpu as pltpu

_ABLATE = os.environ.get("ABLATE", "none")

N_DEV = 4
B = 2
SQ = 128
H_LOC = 4
DH = 64
CHUNK = H_LOC * DH


def kernel(x, Wq, K_ext, V_ext, Wo):
    d_model = x.shape[-1]
    K_ext = K_ext.reshape(B, SQ, CHUNK)
    V_ext = V_ext.reshape(B, SQ, CHUNK)

    def body(x_ref, wq_ref, k_ref, v_ref, wo_ref, out_ref,
             ctx_ref, comm_ref, send_sems, recv_sems):
        my_pos = lax.axis_index("i")

        if _ABLATE == "noattn":
            ctx_ref[:, :, :] = x_ref[:, :, :CHUNK].astype(jnp.bfloat16)
        else:
            xf = jnp.reshape(
                x_ref[:, :, :], (B * SQ, d_model)).astype(jnp.bfloat16)
            wq = wq_ref[:, pl.ds(my_pos * CHUNK, CHUNK)].astype(jnp.bfloat16)
            q = jnp.dot(xf, wq, preferred_element_type=jnp.float32)
            q = q.astype(jnp.bfloat16)

            blocks = []
            for b in range(B):
                kb = k_ref[b, :, :].astype(jnp.bfloat16)
                for h in range(H_LOC):
                    qh = q[b * SQ:(b + 1) * SQ, h * DH:(h + 1) * DH]
                    kh = kb[:, h * DH:(h + 1) * DH]
                    blocks.append(lax.dot_general(
                        qh, kh, (((1,), (1,)), ((), ())),
                        preferred_element_type=jnp.float32,
                    ))
            s = jnp.concatenate(blocks, axis=0) * 0.125
            s = s - jnp.max(s, axis=-1, keepdims=True)
            w = jnp.exp(s)
            w = (w / jnp.sum(w, axis=-1, keepdims=True)).astype(jnp.bfloat16)

            for b in range(B):
                vb = v_ref[b, :, :].astype(jnp.bfloat16)
                for h in range(H_LOC):
                    i = b * H_LOC + h
                    ctx = jnp.dot(
                        w[i * SQ:(i + 1) * SQ, :],
                        vb[:, h * DH:(h + 1) * DH],
                        preferred_element_type=jnp.float32,
                    )
                    ctx_ref[b, :, h * DH:(h + 1) * DH] = ctx.astype(jnp.bfloat16)

        if _ABLATE == "nocomm":
            cf = jnp.reshape(ctx_ref[:, :, :], (B * SQ, CHUNK))
            acc = jnp.dot(
                cf, wo_ref[pl.ds(my_pos * CHUNK, CHUNK), :].astype(jnp.bfloat16),
                preferred_element_type=jnp.float32,
            )
            for s in (0, 2, 1):
                origin = lax.rem(my_pos + s + 1, N_DEV)
                acc = acc + jnp.dot(
                    cf, wo_ref[pl.ds(origin * CHUNK, CHUNK), :].astype(jnp.bfloat16),
                    preferred_element_type=jnp.float32,
                )
            out_ref[:, :, :] = jnp.reshape(acc, (B, SQ, d_model))
            return

        barrier_sem = pltpu.get_barrier_semaphore()
        for j in range(1, N_DEV):
            pl.semaphore_signal(
                barrier_sem, inc=1,
                device_id=(lax.rem(my_pos + j, N_DEV),),
                device_id_type=pl.DeviceIdType.MESH,
            )
        pl.semaphore_wait(barrier_sem, N_DEV - 1)

        sends = []
        for j in range(1, N_DEV):
            r = pltpu.make_async_remote_copy(
                src_ref=ctx_ref,
                dst_ref=comm_ref.at[N_DEV - 1 - j],
                send_sem=send_sems.at[j - 1],
                recv_sem=recv_sems.at[N_DEV - 1 - j],
                device_id=(lax.rem(my_pos + j, N_DEV),),
                device_id_type=pl.DeviceIdType.MESH,
            )
            r.start()
            sends.append(r)

        cf = jnp.reshape(ctx_ref[:, :, :], (B * SQ, CHUNK))
        acc = jnp.dot(
            cf,
            wo_ref[pl.ds(my_pos * CHUNK, CHUNK), :].astype(jnp.bfloat16),
            preferred_element_type=jnp.float32,
        )

        for s in (0, 2, 1):
            recv = pltpu.make_async_remote_copy(
                src_ref=ctx_ref,
                dst_ref=comm_ref.at[s],
                send_sem=send_sems.at[0],
                recv_sem=recv_sems.at[s],
                device_id=(my_pos,),
                device_id_type=pl.DeviceIdType.MESH,
            )
            recv.wait_recv()
            origin = lax.rem(my_pos + s + 1, N_DEV)
            chunk = jnp.reshape(comm_ref[s, :, :, :], (B * SQ, CHUNK))
            acc = acc + jnp.dot(
                chunk,
                wo_ref[pl.ds(origin * CHUNK, CHUNK), :].astype(jnp.bfloat16),
                preferred_element_type=jnp.float32,
            )

        out_ref[:, :, :] = jnp.reshape(acc, (B, SQ, d_model))

        for r in sends:
            r.wait_send()

    return pl.pallas_call(
        body,
        out_shape=jax.ShapeDtypeStruct((B, SQ, d_model), jnp.float32),
        in_specs=[pl.BlockSpec(memory_space=pltpu.VMEM)] * 5,
        out_specs=pl.BlockSpec(memory_space=pltpu.VMEM),
        scratch_shapes=[
            pltpu.VMEM((B, SQ, CHUNK), jnp.bfloat16),
            pltpu.VMEM((N_DEV - 1, B, SQ, CHUNK), jnp.bfloat16),
            pltpu.SemaphoreType.DMA((N_DEV - 1,)),
            pltpu.SemaphoreType.DMA((N_DEV - 1,)),
        ],
        compiler_params=pltpu.CompilerParams(
            collective_id=None if _ABLATE == "nocomm" else 0),
    )(x, Wq, K_ext, V_ext, Wo)
